# Initial kernel scaffold; baseline (speedup 1.0000x reference)
import jax
import jax.numpy as jnp
from jax.experimental import pallas as pl


def kernel(world_pos, mesh_pos, phi, swelling_phi, swelling_phi_rate, node_type, time, mat_param, edge_index, coarse_edge_index, params):
    raise NotImplementedError("write your pallas kernel here")


# speedup vs baseline: 2.1732x; 2.1732x over previous
import functools

import jax
import jax.numpy as jnp
from jax import lax
from jax.experimental import pallas as pl
from jax.experimental.pallas import tpu as pltpu
from jax.experimental.pallas import tpu_sc as plsc

_H = 128
_CH = 64
_NSC = 2
_NTILE = 16
_NW = _NSC * _NTILE
_BM = 512


def _rup(n, m):
    return ((n + m - 1) // m) * m


def _pad_rows(a, n):
    if a.shape[0] == n:
        return a
    pad = jnp.zeros((n - a.shape[0],) + a.shape[1:], a.dtype)
    return jnp.concatenate([a, pad], axis=0)


def _pad_idx(a, n, val):
    if a.shape[0] == n:
        return a
    return jnp.concatenate([a, jnp.full((n - a.shape[0],), val, a.dtype)])


def _ln_in(o, g, b):
    mu = jnp.mean(o, axis=-1, keepdims=True)
    d = o - mu
    var = jnp.mean(d * d, axis=-1, keepdims=True)
    return d * lax.rsqrt(var + 1e-5) * g + b


def _dot(a, b):
    return jnp.dot(a, b, preferred_element_type=jnp.float32)


def _mlp_ln_body(x_ref, w0_ref, b0_ref, w1_ref, b1_ref, g_ref, bb_ref, o_ref):
    h = jnp.maximum(_dot(x_ref[...], w0_ref[...]) + b0_ref[...], 0.0)
    o = _dot(h, w1_ref[...]) + b1_ref[...]
    o_ref[...] = _ln_in(o, g_ref[...], bb_ref[...])


def _tc_mlp_ln(x, w0, b0, w1, b1, g, b):
    n, k = x.shape
    grid = (n // _BM,)
    return pl.pallas_call(
        _mlp_ln_body,
        grid=grid,
        in_specs=[
            pl.BlockSpec((_BM, k), lambda i: (i, 0)),
            pl.BlockSpec((k, _H), lambda i: (0, 0)),
            pl.BlockSpec((1, _H), lambda i: (0, 0)),
            pl.BlockSpec((_H, _H), lambda i: (0, 0)),
            pl.BlockSpec((1, _H), lambda i: (0, 0)),
            pl.BlockSpec((1, _H), lambda i: (0, 0)),
            pl.BlockSpec((1, _H), lambda i: (0, 0)),
        ],
        out_specs=pl.BlockSpec((_BM, _H), lambda i: (i, 0)),
        out_shape=jax.ShapeDtypeStruct((n, _H), jnp.float32),
    )(x, w0, b0, w1, b1, g, b)


def _edge_enc_body(ps_ref, pr_ref, wd_ref, w3_ref, w7_ref, b0_ref, w1_ref,
                   b1_ref, g_ref, bb_ref, o_ref):
    dp = ps_ref[...] - pr_ref[...]
    sq = dp * dp
    li = lax.broadcasted_iota(jnp.int32, dp.shape, 1)
    s1 = jnp.sum(jnp.where(li < 3, sq, 0.0), axis=1, keepdims=True)
    s2 = jnp.sum(jnp.where((li >= 3) & (li < 6), sq, 0.0), axis=1, keepdims=True)
    d = jnp.sqrt(s1 + 1e-12)
    dw = jnp.sqrt(s2 + 1e-12)
    h = jnp.maximum(_dot(dp, wd_ref[...]) + d * w3_ref[...] + dw * w7_ref[...]
                    + b0_ref[...], 0.0)
    o = _dot(h, w1_ref[...]) + b1_ref[...]
    o_ref[...] = _ln_in(o, g_ref[...], bb_ref[...])


def _tc_edge_enc(ps, pr, wd, w3, w7, b0, w1, b1, g, b):
    n = ps.shape[0]
    grid = (n // _BM,)
    row = lambda i: (i, 0)
    full = lambda i: (0, 0)
    return pl.pallas_call(
        _edge_enc_body,
        grid=grid,
        in_specs=[
            pl.BlockSpec((_BM, 16), row),
            pl.BlockSpec((_BM, 16), row),
            pl.BlockSpec((16, _H), full),
            pl.BlockSpec((1, _H), full),
            pl.BlockSpec((1, _H), full),
            pl.BlockSpec((1, _H), full),
            pl.BlockSpec((_H, _H), full),
            pl.BlockSpec((1, _H), full),
            pl.BlockSpec((1, _H), full),
            pl.BlockSpec((1, _H), full),
        ],
        out_specs=pl.BlockSpec((_BM, _H), row),
        out_shape=jax.ShapeDtypeStruct((n, _H), jnp.float32),
    )(ps, pr, wd, w3, w7, b0, w1, b1, g, b)


def _matmul_body(x_ref, w_ref, o_ref):
    o_ref[...] = _dot(x_ref[...], w_ref[...])


def _tc_matmul(x, w):
    n, k = x.shape
    m = w.shape[1]
    grid = (n // _BM,)
    return pl.pallas_call(
        _matmul_body,
        grid=grid,
        in_specs=[
            pl.BlockSpec((_BM, k), lambda i: (i, 0)),
            pl.BlockSpec((k, m), lambda i: (0, 0)),
        ],
        out_specs=pl.BlockSpec((_BM, m), lambda i: (i, 0)),
        out_shape=jax.ShapeDtypeStruct((n, m), jnp.float32),
    )(x, w)


def _edge_step_body(e_ref, gm_ref, gu_ref, w0e_ref, b0_ref, w1_ref, b1_ref,
                    g_ref, bb_ref, msg_ref, en_ref):
    e = e_ref[...]
    ew = _dot(e, w0e_ref[...]) + b0_ref[...]
    hm = jnp.maximum(gm_ref[...] + ew, 0.0)
    hu = jnp.maximum(gu_ref[...] + ew, 0.0)
    g = g_ref[...]
    bb = bb_ref[...]
    w1 = w1_ref[...]
    b1 = b1_ref[...]
    msg_ref[...] = _ln_in(_dot(hm, w1) + b1, g, bb)
    en_ref[...] = _ln_in(_dot(hu, w1) + b1, g, bb) + e


def _tc_edge_step(e, gm, gu, w0e, b0, w1, b1, g, b):
    n = e.shape[0]
    grid = (n // _BM,)
    row = lambda i: (i, 0)
    full = lambda i: (0, 0)
    return pl.pallas_call(
        _edge_step_body,
        grid=grid,
        in_specs=[
            pl.BlockSpec((_BM, _H), row),
            pl.BlockSpec((_BM, _H), row),
            pl.BlockSpec((_BM, _H), row),
            pl.BlockSpec((_H, _H), full),
            pl.BlockSpec((1, _H), full),
            pl.BlockSpec((_H, _H), full),
            pl.BlockSpec((1, _H), full),
            pl.BlockSpec((1, _H), full),
            pl.BlockSpec((1, _H), full),
        ],
        out_specs=[pl.BlockSpec((_BM, _H), row), pl.BlockSpec((_BM, _H), row)],
        out_shape=[jax.ShapeDtypeStruct((n, _H), jnp.float32),
                   jax.ShapeDtypeStruct((n, _H), jnp.float32)],
    )(e, gm, gu, w0e, b0, w1, b1, g, b)


def _node_step_body(p0_ref, p1_ref, x_ref, w0a_ref, w0x_ref, b0_ref, w1_ref,
                    b1_ref, g_ref, bb_ref, o_ref):
    x = x_ref[...]
    agg = p0_ref[...] + p1_ref[...]
    h = jnp.maximum(_dot(agg, w0a_ref[...]) + _dot(x, w0x_ref[...])
                    + b0_ref[...], 0.0)
    o = _dot(h, w1_ref[...]) + b1_ref[...]
    o_ref[...] = _ln_in(o, g_ref[...], bb_ref[...]) + x


def _tc_node_step(p0, p1, x, w0a, w0x, b0, w1, b1, g, b):
    n = x.shape[0]
    grid = (n // _BM,)
    row = lambda i: (i, 0)
    full = lambda i: (0, 0)
    return pl.pallas_call(
        _node_step_body,
        grid=grid,
        in_specs=[
            pl.BlockSpec((_BM, _H), row),
            pl.BlockSpec((_BM, _H), row),
            pl.BlockSpec((_BM, _H), row),
            pl.BlockSpec((_H, _H), full),
            pl.BlockSpec((_H, _H), full),
            pl.BlockSpec((1, _H), full),
            pl.BlockSpec((_H, _H), full),
            pl.BlockSpec((1, _H), full),
            pl.BlockSpec((1, _H), full),
            pl.BlockSpec((1, _H), full),
        ],
        out_specs=pl.BlockSpec((_BM, _H), row),
        out_shape=jax.ShapeDtypeStruct((n, _H), jnp.float32),
    )(p0, p1, x, w0a, w0x, b0, w1, b1, g, b)


def _dec_body(x_ref, cx_ref, w0a_ref, w0x_ref, b0_ref, w1_ref, b1_ref, o_ref):
    h = jnp.maximum(_dot(x_ref[...], w0a_ref[...])
                    + _dot(cx_ref[...], w0x_ref[...]) + b0_ref[...], 0.0)
    o_ref[...] = _dot(h, w1_ref[...]) + b1_ref[...]


def _tc_decoder(x, cx, w0a, w0x, b0, w1, b1):
    n = x.shape[0]
    grid = (n // _BM,)
    row = lambda i: (i, 0)
    full = lambda i: (0, 0)
    return pl.pallas_call(
        _dec_body,
        grid=grid,
        in_specs=[
            pl.BlockSpec((_BM, _H), row),
            pl.BlockSpec((_BM, _H), row),
            pl.BlockSpec((_H, _H), full),
            pl.BlockSpec((_H, _H), full),
            pl.BlockSpec((1, _H), full),
            pl.BlockSpec((_H, _H), full),
            pl.BlockSpec((1, _H), full),
        ],
        out_specs=pl.BlockSpec((_BM, _H), row),
        out_shape=jax.ShapeDtypeStruct((n, _H), jnp.float32),
    )(x, cx, w0a, w0x, b0, w1, b1)


def _sc_mesh():
    return plsc.VectorSubcoreMesh(core_axis_name="c", subcore_axis_name="s")


def _sc_gather_pair(tab, s_idx, r_idx):
    ne = s_idx.shape[0]
    d = tab.shape[1]
    per_tile = ne // _NW
    n_ch = per_tile // _CH

    @functools.partial(
        pl.kernel,
        out_type=(jax.ShapeDtypeStruct((ne, d), jnp.float32),
                  jax.ShapeDtypeStruct((ne, d), jnp.float32)),
        mesh=_sc_mesh(),
        scratch_types=[
            pltpu.VMEM((_CH,), jnp.int32),
            pltpu.VMEM((_CH,), jnp.int32),
            pltpu.VMEM((_CH, d), jnp.float32),
            pltpu.VMEM((_CH, d), jnp.float32),
            pltpu.SemaphoreType.DMA,
        ],
        compiler_params=pltpu.CompilerParams(use_tc_tiling_on_sc=False),
    )
    def k(tab_h, s_h, r_h, ps_h, pr_h, sbuf, rbuf, bs, br, sem):
        w = lax.axis_index("c") * _NTILE + lax.axis_index("s")
        base = w * per_tile

        def body(j, carry):
            off = pl.multiple_of(base + j * _CH, _CH)
            pltpu.sync_copy(s_h.at[pl.ds(off, _CH)], sbuf)
            pltpu.sync_copy(r_h.at[pl.ds(off, _CH)], rbuf)
            cs = pltpu.async_copy(tab_h.at[sbuf], bs, sem)
            cr = pltpu.async_copy(tab_h.at[rbuf], br, sem)
            cs.wait()
            cr.wait()
            pltpu.sync_copy(bs, ps_h.at[pl.ds(off, _CH)])
            pltpu.sync_copy(br, pr_h.at[pl.ds(off, _CH)])
            return carry

        lax.fori_loop(0, n_ch, body, 0)

    return k(tab, s_idx, r_idx)


def _sc_cross_gather(ab, s_idx, r_idx):
    ne = s_idx.shape[0]
    per_tile = ne // _NW
    n_ch = per_tile // _CH

    @functools.partial(
        pl.kernel,
        out_type=(jax.ShapeDtypeStruct((ne, _H), jnp.float32),
                  jax.ShapeDtypeStruct((ne, _H), jnp.float32)),
        mesh=_sc_mesh(),
        scratch_types=[
            pltpu.VMEM((_CH,), jnp.int32),
            pltpu.VMEM((_CH,), jnp.int32),
            pltpu.VMEM((_CH, 2 * _H), jnp.float32),
            pltpu.VMEM((_CH, 2 * _H), jnp.float32),
            pltpu.VMEM((_CH, _H), jnp.float32),
            pltpu.VMEM((_CH, _H), jnp.float32),
            pltpu.SemaphoreType.DMA,
        ],
    )
    def k(ab_h, s_h, r_h, gm_h, gu_h, sbuf, rbuf, ts, tr, gm, gu, sem):
        w = lax.axis_index("c") * _NTILE + lax.axis_index("s")
        base = w * per_tile

        def body(j, carry):
            off = pl.multiple_of(base + j * _CH, _CH)
            pltpu.sync_copy(s_h.at[pl.ds(off, _CH)], sbuf)
            pltpu.sync_copy(r_h.at[pl.ds(off, _CH)], rbuf)
            cs = pltpu.async_copy(ab_h.at[sbuf], ts, sem)
            cr = pltpu.async_copy(ab_h.at[rbuf], tr, sem)
            cs.wait()
            cr.wait()

            def rowfn(i, c2):
                for gidx in range(_H // 16):
                    sl = pl.ds(gidx * 16, 16)
                    sh = pl.ds(_H + gidx * 16, 16)
                    gm[i, sl] = tr[i, sl] + ts[i, sh]
                    gu[i, sl] = ts[i, sl] + tr[i, sh]
                return c2

            lax.fori_loop(0, _CH, rowfn, 0)
            pltpu.sync_copy(gm, gm_h.at[pl.ds(off, _CH)])
            pltpu.sync_copy(gu, gu_h.at[pl.ds(off, _CH)])
            return carry

        lax.fori_loop(0, n_ch, body, 0)

    return k(ab, s_idx, r_idx)


def _sc_segsum(msg, r2d, nacc):
    ne = msg.shape[0]
    per_tile = ne // _NW
    n_ch = per_tile // _CH
    rows_per_tile = nacc // _NTILE
    n_zch = rows_per_tile // _CH

    @functools.partial(
        pl.kernel,
        out_type=jax.ShapeDtypeStruct((_NSC, nacc, _H), jnp.float32),
        mesh=_sc_mesh(),
        scratch_types=[
            pltpu.VMEM((_CH, _H), jnp.float32),
            pltpu.VMEM((_CH, _H), jnp.float32),
            pltpu.VMEM((n_ch, _CH), jnp.int32),
            pltpu.VMEM_SHARED((nacc, _H), jnp.float32),
            pltpu.SemaphoreType.DMA,
        ],
    )
    def k(msg_h, r2d_h, out_h, zbuf, mbuf, idxb, acc, sem):
        c = lax.axis_index("c")
        s = lax.axis_index("s")
        w = c * _NTILE + s

        def zrow(i, carry):
            for gidx in range(_H // 16):
                zbuf[i, pl.ds(gidx * 16, 16)] = jnp.zeros((16,), jnp.float32)
            return carry

        lax.fori_loop(0, _CH, zrow, 0)

        def zc(j, carry):
            pltpu.sync_copy(zbuf, acc.at[pl.ds(s * rows_per_tile + j * _CH, _CH)])
            return carry

        lax.fori_loop(0, n_zch, zc, 0)
        plsc.subcore_barrier()

        pltpu.sync_copy(r2d_h.at[w], idxb)

        def body(j, carry):
            off = pl.multiple_of(w * per_tile + j * _CH, _CH)
            pltpu.sync_copy(msg_h.at[pl.ds(off, _CH)], mbuf)
            pltpu.sync_copy(mbuf, acc.at[idxb.at[j]], add=True)
            return carry

        lax.fori_loop(0, n_ch, body, 0)
        plsc.subcore_barrier()

        def wc(j, carry):
            rows = pl.ds(s * rows_per_tile + j * _CH, _CH)
            pltpu.sync_copy(acc.at[rows], out_h.at[c].at[rows])
            return carry

        lax.fori_loop(0, n_zch, wc, 0)

    return k(msg, r2d)


def _prep_edge_enc(p):
    w0 = p['w0']
    wd = jnp.zeros((16, _H), jnp.float32)
    wd = wd.at[0:3].set(w0[0:3])
    wd = wd.at[3:6].set(w0[4:7])
    wd = wd.at[6].set(w0[8])
    return (wd, w0[3:4], w0[7:8], p['b0'][None, :], p['w1'], p['b1'][None, :],
            p['ln_g'][None, :], p['ln_b'][None, :])


def kernel(world_pos, mesh_pos, phi, swelling_phi, swelling_phi_rate,
           node_type, time, mat_param, edge_index, coarse_edge_index, params):
    f32 = jnp.float32
    n = world_pos.shape[0]
    e = edge_index.shape[1]
    ce = coarse_edge_index.shape[1]
    blk = _NW * _CH
    np_ = _rup(n, blk)
    ep = _rup(e, blk)
    cep = _rup(ce, blk)

    t = time[0]
    freqs = 2.0 ** jnp.arange(8, dtype=f32)
    temb = jnp.concatenate([jnp.sin(freqs * t), jnp.cos(freqs * t)])
    x36 = jnp.concatenate([
        world_pos - mesh_pos, phi, swelling_phi, swelling_phi_rate, node_type,
        jnp.tile(temb[None, :], (n, 1)), jnp.tile(time[None, :], (n, 1)),
        jnp.tile(mat_param[None, :], (n, 1))], axis=1)
    x64 = _pad_rows(jnp.pad(x36, ((0, 0), (0, 64 - x36.shape[1]))), np_)

    p16 = _pad_rows(jnp.pad(
        jnp.concatenate([mesh_pos, world_pos, phi], axis=1),
        ((0, 0), (0, 9))), np_)

    s_f = _pad_idx(edge_index[0], ep, 0)
    r_f = _pad_idx(edge_index[1], ep, 0)
    s_c = _pad_idx(coarse_edge_index[0], cep, 0)
    r_c = _pad_idx(coarse_edge_index[1], cep, 0)
    r2d_f = _pad_idx(edge_index[1], ep, n).reshape(_NW, ep // (_NW * _CH), _CH)
    r2d_c = _pad_idx(coarse_edge_index[1], cep, n).reshape(
        _NW, cep // (_NW * _CH), _CH)

    pp = params
    ne = pp['node_enc']
    w0n = jnp.pad(ne['w0'], ((0, 64 - ne['w0'].shape[0]), (0, 0)))
    x_h = _tc_mlp_ln(x64, w0n, ne['b0'][None, :], ne['w1'], ne['b1'][None, :],
                     ne['ln_g'][None, :], ne['ln_b'][None, :])

    ps, pr = _sc_gather_pair(p16, s_f, r_f)
    e_h = _tc_edge_enc(ps, pr, *_prep_edge_enc(pp['edge_enc']))
    cps, cpr = _sc_gather_pair(p16, s_c, r_c)
    ce_h = _tc_edge_enc(cps, cpr, *_prep_edge_enc(pp['cedge_enc']))

    def run_scale(x_h, e_h, procs, s_idx, r_idx, r2d):
        for p_ in procs:
            em = p_['edge_mlp']
            nm = p_['node_mlp']
            wsr = jnp.concatenate([em['w0'][0:_H, :], em['w0'][_H:2 * _H, :]],
                                  axis=1)
            ab = _tc_matmul(x_h, wsr)
            gm, gu = _sc_cross_gather(ab, s_idx, r_idx)
            msg, e_h = _tc_edge_step(
                e_h, gm, gu, em['w0'][2 * _H:3 * _H, :], em['b0'][None, :],
                em['w1'], em['b1'][None, :], em['ln_g'][None, :],
                em['ln_b'][None, :])
            part = _sc_segsum(msg, r2d, np_)
            x_h = _tc_node_step(
                part[0], part[1], x_h, nm['w0'][0:_H, :], nm['w0'][_H:2 * _H, :],
                nm['b0'][None, :], nm['w1'], nm['b1'][None, :],
                nm['ln_g'][None, :], nm['ln_b'][None, :])
        return x_h, e_h

    cx_h = x_h
    x_h, e_h = run_scale(x_h, e_h, pp['procs'], s_f, r_f, r2d_f)
    cx_h, ce_h = run_scale(cx_h, ce_h, pp['cprocs'], s_c, r_c, r2d_c)

    dec = pp['dec']
    w1p = jnp.pad(dec['w1'], ((0, 0), (0, _H - dec['w1'].shape[1])))
    b1p = jnp.pad(dec['b1'], (0, _H - dec['b1'].shape[0]))[None, :]
    out = _tc_decoder(x_h, cx_h, dec['w0'][0:_H, :], dec['w0'][_H:2 * _H, :],
                      dec['b0'][None, :], w1p, b1p)
    return out[:n, :3]


# speedup vs baseline: 2.5668x; 1.1811x over previous
import functools

import jax
import jax.numpy as jnp
from jax import lax
from jax.experimental import pallas as pl
from jax.experimental.pallas import tpu as pltpu
from jax.experimental.pallas import tpu_sc as plsc

_H = 128
_CH = 64
_NSC = 2
_NTILE = 16
_NW = _NSC * _NTILE
_BM = 512


def _rup(n, m):
    return ((n + m - 1) // m) * m


def _pad_rows(a, n):
    if a.shape[0] == n:
        return a
    pad = jnp.zeros((n - a.shape[0],) + a.shape[1:], a.dtype)
    return jnp.concatenate([a, pad], axis=0)


def _pad_idx(a, n, val):
    if a.shape[0] == n:
        return a
    return jnp.concatenate([a, jnp.full((n - a.shape[0],), val, a.dtype)])


def _ln_in(o, g, b):
    mu = jnp.mean(o, axis=-1, keepdims=True)
    d = o - mu
    var = jnp.mean(d * d, axis=-1, keepdims=True)
    return d * lax.rsqrt(var + 1e-5) * g + b


def _dot(a, b):
    return jnp.dot(a, b, preferred_element_type=jnp.float32)


def _mlp_ln_body(x_ref, w0_ref, b0_ref, w1_ref, b1_ref, g_ref, bb_ref, o_ref):
    h = jnp.maximum(_dot(x_ref[...], w0_ref[...]) + b0_ref[...], 0.0)
    o = _dot(h, w1_ref[...]) + b1_ref[...]
    o_ref[...] = _ln_in(o, g_ref[...], bb_ref[...])


def _tc_mlp_ln(x, w0, b0, w1, b1, g, b):
    n, k = x.shape
    grid = (n // _BM,)
    return pl.pallas_call(
        _mlp_ln_body,
        grid=grid,
        in_specs=[
            pl.BlockSpec((_BM, k), lambda i: (i, 0)),
            pl.BlockSpec((k, _H), lambda i: (0, 0)),
            pl.BlockSpec((1, _H), lambda i: (0, 0)),
            pl.BlockSpec((_H, _H), lambda i: (0, 0)),
            pl.BlockSpec((1, _H), lambda i: (0, 0)),
            pl.BlockSpec((1, _H), lambda i: (0, 0)),
            pl.BlockSpec((1, _H), lambda i: (0, 0)),
        ],
        out_specs=pl.BlockSpec((_BM, _H), lambda i: (i, 0)),
        out_shape=jax.ShapeDtypeStruct((n, _H), jnp.float32),
    )(x, w0, b0, w1, b1, g, b)


def _edge_enc_body(ps_ref, pr_ref, wd_ref, w3_ref, w7_ref, b0_ref, w1_ref,
                   b1_ref, g_ref, bb_ref, o_ref):
    dp = ps_ref[...] - pr_ref[...]
    sq = dp * dp
    li = lax.broadcasted_iota(jnp.int32, dp.shape, 1)
    s1 = jnp.sum(jnp.where(li < 3, sq, 0.0), axis=1, keepdims=True)
    s2 = jnp.sum(jnp.where((li >= 3) & (li < 6), sq, 0.0), axis=1, keepdims=True)
    d = jnp.sqrt(s1 + 1e-12)
    dw = jnp.sqrt(s2 + 1e-12)
    h = jnp.maximum(_dot(dp, wd_ref[...]) + d * w3_ref[...] + dw * w7_ref[...]
                    + b0_ref[...], 0.0)
    o = _dot(h, w1_ref[...]) + b1_ref[...]
    o_ref[...] = _ln_in(o, g_ref[...], bb_ref[...])


def _tc_edge_enc(ps, pr, wd, w3, w7, b0, w1, b1, g, b):
    n = ps.shape[0]
    grid = (n // _BM,)
    row = lambda i: (i, 0)
    full = lambda i: (0, 0)
    return pl.pallas_call(
        _edge_enc_body,
        grid=grid,
        in_specs=[
            pl.BlockSpec((_BM, 16), row),
            pl.BlockSpec((_BM, 16), row),
            pl.BlockSpec((16, _H), full),
            pl.BlockSpec((1, _H), full),
            pl.BlockSpec((1, _H), full),
            pl.BlockSpec((1, _H), full),
            pl.BlockSpec((_H, _H), full),
            pl.BlockSpec((1, _H), full),
            pl.BlockSpec((1, _H), full),
            pl.BlockSpec((1, _H), full),
        ],
        out_specs=pl.BlockSpec((_BM, _H), row),
        out_shape=jax.ShapeDtypeStruct((n, _H), jnp.float32),
    )(ps, pr, wd, w3, w7, b0, w1, b1, g, b)


def _matmul_body(x_ref, w_ref, o_ref):
    o_ref[...] = _dot(x_ref[...], w_ref[...])


def _tc_matmul(x, w):
    n, k = x.shape
    m = w.shape[1]
    grid = (n // _BM,)
    return pl.pallas_call(
        _matmul_body,
        grid=grid,
        in_specs=[
            pl.BlockSpec((_BM, k), lambda i: (i, 0)),
            pl.BlockSpec((k, m), lambda i: (0, 0)),
        ],
        out_specs=pl.BlockSpec((_BM, m), lambda i: (i, 0)),
        out_shape=jax.ShapeDtypeStruct((n, m), jnp.float32),
    )(x, w)


def _edge_step_body(e_ref, gm_ref, gu_ref, w0e_ref, b0_ref, w1_ref, b1_ref,
                    g_ref, bb_ref, msg_ref, en_ref):
    e = e_ref[...]
    ew = _dot(e, w0e_ref[...]) + b0_ref[...]
    hm = jnp.maximum(gm_ref[...] + ew, 0.0)
    hu = jnp.maximum(gu_ref[...] + ew, 0.0)
    g = g_ref[...]
    bb = bb_ref[...]
    w1 = w1_ref[...]
    b1 = b1_ref[...]
    msg_ref[...] = _ln_in(_dot(hm, w1) + b1, g, bb)
    en_ref[...] = _ln_in(_dot(hu, w1) + b1, g, bb) + e


def _tc_edge_step(e, gm, gu, w0e, b0, w1, b1, g, b):
    n = e.shape[0]
    grid = (n // _BM,)
    row = lambda i: (i, 0)
    full = lambda i: (0, 0)
    return pl.pallas_call(
        _edge_step_body,
        grid=grid,
        in_specs=[
            pl.BlockSpec((_BM, _H), row),
            pl.BlockSpec((_BM, _H), row),
            pl.BlockSpec((_BM, _H), row),
            pl.BlockSpec((_H, _H), full),
            pl.BlockSpec((1, _H), full),
            pl.BlockSpec((_H, _H), full),
            pl.BlockSpec((1, _H), full),
            pl.BlockSpec((1, _H), full),
            pl.BlockSpec((1, _H), full),
        ],
        out_specs=[pl.BlockSpec((_BM, _H), row), pl.BlockSpec((_BM, _H), row)],
        out_shape=[jax.ShapeDtypeStruct((n, _H), jnp.float32),
                   jax.ShapeDtypeStruct((n, _H), jnp.float32)],
    )(e, gm, gu, w0e, b0, w1, b1, g, b)


def _node_step_body(p0_ref, p1_ref, x_ref, w0a_ref, w0x_ref, b0_ref, w1_ref,
                    b1_ref, g_ref, bb_ref, o_ref):
    x = x_ref[...]
    agg = p0_ref[...] + p1_ref[...]
    h = jnp.maximum(_dot(agg, w0a_ref[...]) + _dot(x, w0x_ref[...])
                    + b0_ref[...], 0.0)
    o = _dot(h, w1_ref[...]) + b1_ref[...]
    o_ref[...] = _ln_in(o, g_ref[...], bb_ref[...]) + x


def _tc_node_step(p0, p1, x, w0a, w0x, b0, w1, b1, g, b):
    n = x.shape[0]
    grid = (n // _BM,)
    row = lambda i: (i, 0)
    full = lambda i: (0, 0)
    return pl.pallas_call(
        _node_step_body,
        grid=grid,
        in_specs=[
            pl.BlockSpec((_BM, _H), row),
            pl.BlockSpec((_BM, _H), row),
            pl.BlockSpec((_BM, _H), row),
            pl.BlockSpec((_H, _H), full),
            pl.BlockSpec((_H, _H), full),
            pl.BlockSpec((1, _H), full),
            pl.BlockSpec((_H, _H), full),
            pl.BlockSpec((1, _H), full),
            pl.BlockSpec((1, _H), full),
            pl.BlockSpec((1, _H), full),
        ],
        out_specs=pl.BlockSpec((_BM, _H), row),
        out_shape=jax.ShapeDtypeStruct((n, _H), jnp.float32),
    )(p0, p1, x, w0a, w0x, b0, w1, b1, g, b)


def _dec_body(x_ref, cx_ref, w0a_ref, w0x_ref, b0_ref, w1_ref, b1_ref, o_ref):
    h = jnp.maximum(_dot(x_ref[...], w0a_ref[...])
                    + _dot(cx_ref[...], w0x_ref[...]) + b0_ref[...], 0.0)
    o_ref[...] = _dot(h, w1_ref[...]) + b1_ref[...]


def _tc_decoder(x, cx, w0a, w0x, b0, w1, b1):
    n = x.shape[0]
    grid = (n // _BM,)
    row = lambda i: (i, 0)
    full = lambda i: (0, 0)
    return pl.pallas_call(
        _dec_body,
        grid=grid,
        in_specs=[
            pl.BlockSpec((_BM, _H), row),
            pl.BlockSpec((_BM, _H), row),
            pl.BlockSpec((_H, _H), full),
            pl.BlockSpec((_H, _H), full),
            pl.BlockSpec((1, _H), full),
            pl.BlockSpec((_H, _H), full),
            pl.BlockSpec((1, _H), full),
        ],
        out_specs=pl.BlockSpec((_BM, _H), row),
        out_shape=jax.ShapeDtypeStruct((n, _H), jnp.float32),
    )(x, cx, w0a, w0x, b0, w1, b1)


def _sc_mesh():
    return plsc.VectorSubcoreMesh(core_axis_name="c", subcore_axis_name="s")


def _sc_gather_pair(tab, s_idx, r_idx):
    ne = s_idx.shape[0]
    d = tab.shape[1]
    per_tile = ne // _NW
    n_ch = per_tile // _CH

    @functools.partial(
        pl.kernel,
        out_type=(jax.ShapeDtypeStruct((ne, d), jnp.float32),
                  jax.ShapeDtypeStruct((ne, d), jnp.float32)),
        mesh=_sc_mesh(),
        scratch_types=[
            pltpu.VMEM((_CH,), jnp.int32),
            pltpu.VMEM((_CH,), jnp.int32),
            pltpu.VMEM((_CH, d), jnp.float32),
            pltpu.VMEM((_CH, d), jnp.float32),
            pltpu.SemaphoreType.DMA,
        ],
        compiler_params=pltpu.CompilerParams(use_tc_tiling_on_sc=False),
    )
    def k(tab_h, s_h, r_h, ps_h, pr_h, sbuf, rbuf, bs, br, sem):
        w = lax.axis_index("c") * _NTILE + lax.axis_index("s")
        base = w * per_tile

        def body(j, carry):
            off = pl.multiple_of(base + j * _CH, _CH)
            pltpu.sync_copy(s_h.at[pl.ds(off, _CH)], sbuf)
            pltpu.sync_copy(r_h.at[pl.ds(off, _CH)], rbuf)
            cs = pltpu.async_copy(tab_h.at[sbuf], bs, sem)
            cr = pltpu.async_copy(tab_h.at[rbuf], br, sem)
            cs.wait()
            cr.wait()
            pltpu.sync_copy(bs, ps_h.at[pl.ds(off, _CH)])
            pltpu.sync_copy(br, pr_h.at[pl.ds(off, _CH)])
            return carry

        lax.fori_loop(0, n_ch, body, 0)

    return k(tab, s_idx, r_idx)


def _sc_cross_gather(ab, s2d, r2d):
    n_ch = s2d.shape[1]
    ne = _NW * n_ch * _CH
    per_tile = n_ch * _CH
    assert n_ch % 2 == 0

    @functools.partial(
        pl.kernel,
        out_type=(jax.ShapeDtypeStruct((ne, _H), jnp.float32),
                  jax.ShapeDtypeStruct((ne, _H), jnp.float32)),
        mesh=_sc_mesh(),
        scratch_types=[
            pltpu.VMEM((n_ch, _CH), jnp.int32),
            pltpu.VMEM((n_ch, _CH), jnp.int32),
            pltpu.VMEM((2, _CH, 2 * _H), jnp.float32),
            pltpu.VMEM((2, _CH, 2 * _H), jnp.float32),
            pltpu.VMEM((_CH, _H), jnp.float32),
            pltpu.VMEM((_CH, _H), jnp.float32),
            pltpu.SemaphoreType.DMA,
            pltpu.SemaphoreType.DMA,
        ],
    )
    def k(ab_h, s_h, r_h, gm_h, gu_h, sidx, ridx, ts, tr, gm, gu, sem0, sem1):
        w = lax.axis_index("c") * _NTILE + lax.axis_index("s")
        base = w * per_tile
        sems = (sem0, sem1)

        pltpu.sync_copy(s_h.at[w], sidx)
        pltpu.sync_copy(r_h.at[w], ridx)

        def fire(j, slot):
            pltpu.async_copy(ab_h.at[sidx.at[j]], ts.at[slot], sems[slot])
            pltpu.async_copy(ab_h.at[ridx.at[j]], tr.at[slot], sems[slot])

        def drain(j, slot):
            pltpu.make_async_copy(ab_h.at[sidx.at[j]], ts.at[slot],
                                  sems[slot]).wait()
            pltpu.make_async_copy(ab_h.at[ridx.at[j]], tr.at[slot],
                                  sems[slot]).wait()

        def compute_out(j, slot):
            def rowfn(i, c2):
                for gidx in range(_H // 16):
                    sl = pl.ds(gidx * 16, 16)
                    sh = pl.ds(_H + gidx * 16, 16)
                    gm[i, sl] = tr[slot, i, sl] + ts[slot, i, sh]
                    gu[i, sl] = ts[slot, i, sl] + tr[slot, i, sh]
                return c2

            lax.fori_loop(0, _CH, rowfn, 0)
            off = pl.multiple_of(base + j * _CH, _CH)
            pltpu.sync_copy(gm, gm_h.at[pl.ds(off, _CH)])
            pltpu.sync_copy(gu, gu_h.at[pl.ds(off, _CH)])

        fire(0, 0)

        def body(i, carry):
            j0 = 2 * i
            fire(j0 + 1, 1)
            drain(j0, 0)
            compute_out(j0, 0)

            @pl.when(i + 1 < n_ch // 2)
            def _():
                fire(j0 + 2, 0)

            drain(j0 + 1, 1)
            compute_out(j0 + 1, 1)
            return carry

        lax.fori_loop(0, n_ch // 2, body, 0)

    return k(ab, s2d, r2d)


def _sc_segsum(msg, r2d, nacc):
    ne = msg.shape[0]
    per_tile = ne // _NW
    n_ch = per_tile // _CH
    rows_per_tile = nacc // _NTILE
    n_zch = rows_per_tile // _CH

    @functools.partial(
        pl.kernel,
        out_type=jax.ShapeDtypeStruct((_NSC, nacc, _H), jnp.float32),
        mesh=_sc_mesh(),
        scratch_types=[
            pltpu.VMEM((_CH, _H), jnp.float32),
            pltpu.VMEM((_CH, _H), jnp.float32),
            pltpu.VMEM((n_ch, _CH), jnp.int32),
            pltpu.VMEM_SHARED((nacc, _H), jnp.float32),
            pltpu.SemaphoreType.DMA,
        ],
    )
    def k(msg_h, r2d_h, out_h, zbuf, mbuf, idxb, acc, sem):
        c = lax.axis_index("c")
        s = lax.axis_index("s")
        w = c * _NTILE + s

        def zrow(i, carry):
            for gidx in range(_H // 16):
                zbuf[i, pl.ds(gidx * 16, 16)] = jnp.zeros((16,), jnp.float32)
            return carry

        lax.fori_loop(0, _CH, zrow, 0)

        def zc(j, carry):
            pltpu.sync_copy(zbuf, acc.at[pl.ds(s * rows_per_tile + j * _CH, _CH)])
            return carry

        lax.fori_loop(0, n_zch, zc, 0)
        plsc.subcore_barrier()

        pltpu.sync_copy(r2d_h.at[w], idxb)

        def body(j, carry):
            off = pl.multiple_of(w * per_tile + j * _CH, _CH)
            pltpu.sync_copy(msg_h.at[pl.ds(off, _CH)], mbuf)
            pltpu.sync_copy(mbuf, acc.at[idxb.at[j]], add=True)
            return carry

        lax.fori_loop(0, n_ch, body, 0)
        plsc.subcore_barrier()

        def wc(j, carry):
            rows = pl.ds(s * rows_per_tile + j * _CH, _CH)
            pltpu.sync_copy(acc.at[rows], out_h.at[c].at[rows])
            return carry

        lax.fori_loop(0, n_zch, wc, 0)

    return k(msg, r2d)


def _prep_edge_enc(p):
    w0 = p['w0']
    wd = jnp.zeros((16, _H), jnp.float32)
    wd = wd.at[0:3].set(w0[0:3])
    wd = wd.at[3:6].set(w0[4:7])
    wd = wd.at[6].set(w0[8])
    return (wd, w0[3:4], w0[7:8], p['b0'][None, :], p['w1'], p['b1'][None, :],
            p['ln_g'][None, :], p['ln_b'][None, :])


def kernel(world_pos, mesh_pos, phi, swelling_phi, swelling_phi_rate,
           node_type, time, mat_param, edge_index, coarse_edge_index, params):
    f32 = jnp.float32
    n = world_pos.shape[0]
    e = edge_index.shape[1]
    ce = coarse_edge_index.shape[1]
    blk = _NW * _CH * 2
    np_ = _rup(n, _NW * _CH)
    ep = _rup(e, blk)
    cep = _rup(ce, blk)

    t = time[0]
    freqs = 2.0 ** jnp.arange(8, dtype=f32)
    temb = jnp.concatenate([jnp.sin(freqs * t), jnp.cos(freqs * t)])
    x36 = jnp.concatenate([
        world_pos - mesh_pos, phi, swelling_phi, swelling_phi_rate, node_type,
        jnp.tile(temb[None, :], (n, 1)), jnp.tile(time[None, :], (n, 1)),
        jnp.tile(mat_param[None, :], (n, 1))], axis=1)
    x64 = _pad_rows(jnp.pad(x36, ((0, 0), (0, 64 - x36.shape[1]))), np_)

    p16 = _pad_rows(jnp.pad(
        jnp.concatenate([mesh_pos, world_pos, phi], axis=1),
        ((0, 0), (0, 9))), np_)

    s_f = _pad_idx(edge_index[0], ep, 0)
    r_f = _pad_idx(edge_index[1], ep, 0)
    s_c = _pad_idx(coarse_edge_index[0], cep, 0)
    r_c = _pad_idx(coarse_edge_index[1], cep, 0)
    sh_f = (_NW, ep // (_NW * _CH), _CH)
    sh_c = (_NW, cep // (_NW * _CH), _CH)
    s2d_f = s_f.reshape(sh_f)
    r2d_f = r_f.reshape(sh_f)
    s2d_c = s_c.reshape(sh_c)
    r2d_c = r_c.reshape(sh_c)
    rsc_f = _pad_idx(edge_index[1], ep, n).reshape(sh_f)
    rsc_c = _pad_idx(coarse_edge_index[1], cep, n).reshape(sh_c)

    pp = params
    ne = pp['node_enc']
    w0n = jnp.pad(ne['w0'], ((0, 64 - ne['w0'].shape[0]), (0, 0)))
    x_h = _tc_mlp_ln(x64, w0n, ne['b0'][None, :], ne['w1'], ne['b1'][None, :],
                     ne['ln_g'][None, :], ne['ln_b'][None, :])

    ps, pr = _sc_gather_pair(p16, s_f, r_f)
    e_h = _tc_edge_enc(ps, pr, *_prep_edge_enc(pp['edge_enc']))
    cps, cpr = _sc_gather_pair(p16, s_c, r_c)
    ce_h = _tc_edge_enc(cps, cpr, *_prep_edge_enc(pp['cedge_enc']))

    def run_scale(x_h, e_h, procs, s2d, r2d, rsc):
        for p_ in procs:
            em = p_['edge_mlp']
            nm = p_['node_mlp']
            wsr = jnp.concatenate([em['w0'][0:_H, :], em['w0'][_H:2 * _H, :]],
                                  axis=1)
            ab = _tc_matmul(x_h, wsr)
            gm, gu = _sc_cross_gather(ab, s2d, r2d)
            msg, e_h = _tc_edge_step(
                e_h, gm, gu, em['w0'][2 * _H:3 * _H, :], em['b0'][None, :],
                em['w1'], em['b1'][None, :], em['ln_g'][None, :],
                em['ln_b'][None, :])
            part = _sc_segsum(msg, rsc, np_)
            x_h = _tc_node_step(
                part[0], part[1], x_h, nm['w0'][0:_H, :], nm['w0'][_H:2 * _H, :],
                nm['b0'][None, :], nm['w1'], nm['b1'][None, :],
                nm['ln_g'][None, :], nm['ln_b'][None, :])
        return x_h, e_h

    cx_h = x_h
    x_h, e_h = run_scale(x_h, e_h, pp['procs'], s2d_f, r2d_f, rsc_f)
    cx_h, ce_h = run_scale(cx_h, ce_h, pp['cprocs'], s2d_c, r2d_c, rsc_c)

    dec = pp['dec']
    w1p = jnp.pad(dec['w1'], ((0, 0), (0, _H - dec['w1'].shape[1])))
    b1p = jnp.pad(dec['b1'], (0, _H - dec['b1'].shape[0]))[None, :]
    out = _tc_decoder(x_h, cx_h, dec['w0'][0:_H, :], dec['w0'][_H:2 * _H, :],
                      dec['b0'][None, :], w1p, b1p)
    return out[:n, :3]
